# baseline (speedup 1.0000x reference)
import functools

import jax
import jax.numpy as jnp
from jax import lax
from jax.experimental import pallas as pl
from jax.experimental.pallas import tpu as pltpu
from jax.experimental.pallas import tpu_sc as plsc

D_MODEL = 64
N_I = 4096
N_J = 200
NUM_WORKERS = 32
CHUNK = 128
NBUF = 4
TBUF = 2
LEAD = 2
SCALE = 8.0
LANES = 16


def _embed_body(xt_hbm, lut2_hbm, out_hbm, idx_v, p_v, pair_v, tr_v,
                gsems, ssems):
    c = lax.axis_index("c")
    s = lax.axis_index("s")
    wid = s * 2 + c
    i0 = wid * CHUNK

    pltpu.sync_copy(xt_hbm.at[:, pl.ds(i0, CHUNK)], idx_v)

    def prep_and_gather(j, b):
        def prep(q, carry):
            p_v[b, pl.ds(LANES * q, LANES)] = (
                idx_v[j, pl.ds(LANES * q, LANES)] >> 1
            )
            return carry

        lax.fori_loop(0, CHUNK // LANES, prep, 0, unroll=8)
        pltpu.async_copy(lut2_hbm.at[p_v.at[b]], pair_v.at[b], gsems[b])

    def wait_gather(b):
        pltpu.make_async_copy(lut2_hbm.at[p_v.at[0]], pair_v.at[b],
                              gsems[b]).wait()

    def start_store(j, t):
        pltpu.async_copy(tr_v.at[t], out_hbm.at[j, :, pl.ds(i0, CHUNK)],
                         ssems[t])

    def wait_store(t):
        pltpu.make_async_copy(tr_v.at[t], out_hbm.at[0, :, pl.ds(i0, CHUNK)],
                              ssems[t]).wait()

    def transpose_scale(j, b, t):
        rows = []
        cols = []
        for q in range(CHUNK // LANES):
            rows.append(
                lax.broadcasted_iota(jnp.int32, (LANES,), 0) + LANES * q
            )
            cols.append(
                (idx_v[j, pl.ds(LANES * q, LANES)] & 1) << 6
            )

        def per_d(d, cols_c):
            new_cols = []
            for q in range(CHUNK // LANES):
                vals = plsc.load_gather(pair_v.at[b], [rows[q], cols_c[q]])
                tr_v[t, d, pl.ds(LANES * q, LANES)] = vals * SCALE
                new_cols.append(cols_c[q] + 1)
            return tuple(new_cols)

        lax.fori_loop(0, D_MODEL, per_d, tuple(cols), unroll=2)

    for j in range(NBUF):
        prep_and_gather(j, j)
    for j in range(LEAD):
        wait_gather(j)
        transpose_scale(j, j, j % TBUF)
        start_store(j, j % TBUF)

    def ring_iter(i, carry):
        for b in range(NBUF):
            j = i * NBUF + b + LEAD
            bj = (b + LEAD) % NBUF
            prep_and_gather(j + LEAD, b)
            wait_gather(bj)
            t = b % TBUF
            wait_store(t)
            transpose_scale(j, bj, t)
            start_store(j, t)
        return carry

    lax.fori_loop(0, (N_J - 2 * LEAD) // NBUF, ring_iter, 0)

    for jj in range(N_J - LEAD, N_J):
        b = jj % NBUF
        wait_gather(b)
        t = jj % TBUF
        wait_store(t)
        transpose_scale(jj, b, t)
        start_store(jj, t)

    for t in range(TBUF):
        wait_store(t)


@jax.jit
def _embed(xt, lut2):
    mesh = plsc.VectorSubcoreMesh(core_axis_name="c", subcore_axis_name="s")
    kern = functools.partial(
        pl.kernel,
        out_type=jax.ShapeDtypeStruct((N_J, D_MODEL, N_I), jnp.float32),
        mesh=mesh,
        compiler_params=pltpu.CompilerParams(needs_layout_passes=False),
        scratch_types=[
            pltpu.VMEM((N_J, CHUNK), jnp.int32),
            pltpu.VMEM((NBUF, CHUNK), jnp.int32),
            pltpu.VMEM((NBUF, CHUNK, 128), jnp.float32),
            pltpu.VMEM((TBUF, D_MODEL, CHUNK), jnp.float32),
            [pltpu.SemaphoreType.DMA] * NBUF,
            [pltpu.SemaphoreType.DMA] * TBUF,
        ],
    )(_embed_body)
    return kern(xt, lut2)


def kernel(x, lut):
    xt = x.astype(jnp.int32).T
    lut2 = lut.reshape(500000, 128)
    out_t = _embed(xt, lut2)
    return out_t.transpose(2, 0, 1)


# speedup vs baseline: 1.0304x; 1.0304x over previous
import functools

import jax
import jax.numpy as jnp
from jax import lax
from jax.experimental import pallas as pl
from jax.experimental.pallas import tpu as pltpu
from jax.experimental.pallas import tpu_sc as plsc

D_MODEL = 64
VOCAB = 1000000
N_I = 4096
N_J = 200
NUM_WORKERS = 32
LANES = 16
SCALE = 8.0

FV = 256
N_FULL = VOCAB // FV
F_TAIL = VOCAB - N_FULL * FV
F_BASE = N_FULL // NUM_WORKERS
F_EXTRA = N_FULL - F_BASE * NUM_WORKERS
F_STEPS = F_BASE + 1

CHUNK = 128
NBUF = 4
TBUF = 2
LEAD = 2


def _iota16():
    return lax.broadcasted_iota(jnp.int32, (LANES,), 0)


def _format_body(lutt_hbm, tail_hbm, tab_hbm, a_v, b_v, t_v, lsems, ssems):
    c = lax.axis_index("c")
    s = lax.axis_index("s")
    wid = s * 2 + c
    base = wid * F_BASE + jnp.minimum(wid, F_EXTRA)

    iota = _iota16()
    hcol = (iota & 1) << 6

    def clamp(k):
        return jnp.minimum(base + k, N_FULL - 1)

    def start_load(k, ab):
        pltpu.async_copy(lutt_hbm.at[:, pl.ds(clamp(k) * FV, FV)],
                         a_v.at[ab], lsems[ab])

    def wait_load(ab):
        pltpu.make_async_copy(lutt_hbm.at[:, pl.ds(0, FV)], a_v.at[ab],
                              lsems[ab]).wait()

    def start_store(k, ab):
        pltpu.async_copy(b_v.at[ab],
                         tab_hbm.at[pl.ds(clamp(k) * (FV // 2), FV // 2)],
                         ssems[ab])

    def wait_store(ab):
        pltpu.make_async_copy(b_v.at[ab], tab_hbm.at[pl.ds(0, FV // 2)],
                              ssems[ab]).wait()

    def transpose_slab(src, dst, nv):
        def per_d(d0, carry):
            dvec = (d0 + iota) & 63
            col = hcol + dvec
            for g in range(nv // LANES):
                vvec = iota + (LANES * g)
                kvec = (iota >> 1) + (8 * g)
                vals = plsc.load_gather(src, [dvec, vvec])
                plsc.store_scatter(dst, [kvec, col], vals * SCALE)
            return carry

        lax.fori_loop(0, D_MODEL, per_d, 0)

    def step(k, ab, first, load_next):
        wait_load(ab)
        if not first:
            wait_store(ab)
        transpose_slab(a_v.at[ab], b_v.at[ab], FV)
        start_store(k, ab)
        if load_next:
            start_load(k + 2, ab)

    start_load(0, 0)
    start_load(1, 1)
    step(0, 0, True, True)
    step(1, 1, True, True)

    def pair_iter(i, carry):
        step(2 * i + 2, 0, False, True)
        step(2 * i + 3, 1, False, True)
        return carry

    lax.fori_loop(0, (F_STEPS - 5) // 2, pair_iter, 0)
    step(F_STEPS - 3, 0, False, True)
    step(F_STEPS - 2, 1, False, False)
    step(F_STEPS - 1, 0, False, False)
    wait_store(0)
    wait_store(1)

    @pl.when(wid == F_EXTRA)
    def _tail():
        pltpu.sync_copy(tail_hbm, t_v)
        transpose_slab(t_v, b_v.at[0], F_TAIL)
        pltpu.async_copy(b_v.at[0, pl.ds(0, F_TAIL // 2)],
                         tab_hbm.at[pl.ds(N_FULL * FV // 2, F_TAIL // 2)],
                         ssems[0])
        pltpu.make_async_copy(b_v.at[0, pl.ds(0, F_TAIL // 2)],
                              tab_hbm.at[pl.ds(0, F_TAIL // 2)],
                              ssems[0]).wait()


def _gather_body(xt_hbm, tab_hbm, out_hbm, idx_v, p_v, h_v, pair_v, tr_v,
                 gsems, ssems):
    c = lax.axis_index("c")
    s = lax.axis_index("s")
    wid = s * 2 + c
    i0 = wid * CHUNK

    iota = _iota16()

    pltpu.sync_copy(xt_hbm.at[:, pl.ds(i0, CHUNK)], idx_v)

    def prep_and_gather(j, b):
        def prep(q, carry):
            xq = idx_v[j, pl.ds(LANES * q, LANES)]
            p_v[b, pl.ds(LANES * q, LANES)] = xq >> 1
            h_v[b, pl.ds(LANES * q, LANES)] = (xq & 1) << 6
            return carry

        lax.fori_loop(0, CHUNK // LANES, prep, 0, unroll=8)
        pltpu.async_copy(tab_hbm.at[p_v.at[b]], pair_v.at[b], gsems[b])

    def wait_gather(b):
        pltpu.make_async_copy(tab_hbm.at[p_v.at[0]], pair_v.at[b],
                              gsems[b]).wait()

    def start_store(j, t):
        pltpu.async_copy(tr_v.at[t], out_hbm.at[j, :, pl.ds(i0, CHUNK)],
                         ssems[t])

    def wait_store(t):
        pltpu.make_async_copy(tr_v.at[t], out_hbm.at[0, :, pl.ds(i0, CHUNK)],
                              ssems[t]).wait()

    def transpose_select(b, t):
        def per_r(r0, carry):
            for q in range(D_MODEL // LANES):
                dq = iota + (LANES * q)
                rr = (r0 + dq) & 127
                h64 = plsc.load_gather(h_v.at[b], [rr])
                vals = plsc.load_gather(pair_v.at[b], [rr, h64 + dq])
                plsc.store_scatter(tr_v.at[t], [dq, rr], vals)
            return carry

        lax.fori_loop(0, CHUNK, per_r, 0)

    for j in range(NBUF):
        prep_and_gather(j, j)
    for j in range(LEAD):
        wait_gather(j)
        transpose_select(j, j % TBUF)
        start_store(j, j % TBUF)

    def ring_iter(i, carry):
        for b in range(NBUF):
            j = i * NBUF + b + LEAD
            bj = (b + LEAD) % NBUF
            prep_and_gather(j + LEAD, b)
            wait_gather(bj)
            t = b % TBUF
            wait_store(t)
            transpose_select(bj, t)
            start_store(j, t)
        return carry

    lax.fori_loop(0, (N_J - 2 * LEAD) // NBUF, ring_iter, 0)

    for jj in range(N_J - LEAD, N_J):
        b = jj % NBUF
        wait_gather(b)
        t = jj % TBUF
        wait_store(t)
        transpose_select(b, t)
        start_store(jj, t)

    for t in range(TBUF):
        wait_store(t)


@jax.jit
def _embed(xt, lutt, tail):
    mesh = plsc.VectorSubcoreMesh(core_axis_name="c", subcore_axis_name="s")
    fmt = functools.partial(
        pl.kernel,
        out_type=jax.ShapeDtypeStruct((VOCAB // 2, 128), jnp.float32),
        mesh=mesh,
        compiler_params=pltpu.CompilerParams(needs_layout_passes=False),
        scratch_types=[
            pltpu.VMEM((2, D_MODEL, FV), jnp.float32),
            pltpu.VMEM((2, FV // 2, 128), jnp.float32),
            pltpu.VMEM((D_MODEL, F_TAIL), jnp.float32),
            [pltpu.SemaphoreType.DMA] * 2,
            [pltpu.SemaphoreType.DMA] * 2,
        ],
    )(_format_body)
    tab = fmt(lutt, tail)

    gat = functools.partial(
        pl.kernel,
        out_type=jax.ShapeDtypeStruct((N_J, D_MODEL, N_I), jnp.float32),
        mesh=mesh,
        compiler_params=pltpu.CompilerParams(needs_layout_passes=False),
        scratch_types=[
            pltpu.VMEM((N_J, CHUNK), jnp.int32),
            pltpu.VMEM((NBUF, CHUNK), jnp.int32),
            pltpu.VMEM((NBUF, CHUNK), jnp.int32),
            pltpu.VMEM((NBUF, CHUNK, 128), jnp.float32),
            pltpu.VMEM((TBUF, D_MODEL, CHUNK), jnp.float32),
            [pltpu.SemaphoreType.DMA] * NBUF,
            [pltpu.SemaphoreType.DMA] * TBUF,
        ],
    )(_gather_body)
    return gat(xt, tab)


def kernel(x, lut):
    xt = x.astype(jnp.int32).T
    lutt = lut.T
    tail = lutt[:, N_FULL * FV:]
    out_t = _embed(xt, lutt, tail)
    return out_t.transpose(2, 0, 1)


# speedup vs baseline: 1.1150x; 1.0822x over previous
import functools

import jax
import jax.numpy as jnp
from jax import lax
from jax.experimental import pallas as pl
from jax.experimental.pallas import tpu as pltpu
from jax.experimental.pallas import tpu_sc as plsc

D_MODEL = 64
VOCAB = 1000000
N_I = 4096
N_J = 200
NUM_WORKERS = 32
LANES = 16
SCALE = 8.0

FV = 256
N_FULL = VOCAB // FV
F_TAIL = VOCAB - N_FULL * FV
F_BASE = N_FULL // NUM_WORKERS
F_EXTRA = N_FULL - F_BASE * NUM_WORKERS
F_STEPS = F_BASE + 1

CHUNK = 128
NBUF = 4
TBUF = 2
LEAD = 2


def _iota16():
    return lax.broadcasted_iota(jnp.int32, (LANES,), 0)


def _format_body(lutt_hbm, tail_hbm, tab_hbm, a_v, b_v, t_v, lsems, ssems):
    c = lax.axis_index("c")
    s = lax.axis_index("s")
    wid = s * 2 + c
    base = wid * F_BASE + jnp.minimum(wid, F_EXTRA)

    iota = _iota16()
    hcol = (iota & 1) << 6

    def clamp(k):
        return jnp.minimum(base + k, N_FULL - 1)

    def start_load(k, ab):
        pltpu.async_copy(lutt_hbm.at[:, pl.ds(clamp(k) * FV, FV)],
                         a_v.at[ab], lsems[ab])

    def wait_load(ab):
        pltpu.make_async_copy(lutt_hbm.at[:, pl.ds(0, FV)], a_v.at[ab],
                              lsems[ab]).wait()

    def start_store(k, ab):
        pltpu.async_copy(b_v.at[ab],
                         tab_hbm.at[pl.ds(clamp(k) * (FV // 2), FV // 2)],
                         ssems[ab])

    def wait_store(ab):
        pltpu.make_async_copy(b_v.at[ab], tab_hbm.at[pl.ds(0, FV // 2)],
                              ssems[ab]).wait()

    def transpose_slab(src, dst, nv):
        def per_d(d0, carry):
            dvec = (d0 + iota) & 63
            col = hcol + dvec
            for g in range(nv // LANES):
                vvec = iota + (LANES * g)
                kvec = (iota >> 1) + (8 * g)
                vals = plsc.load_gather(src, [dvec, vvec])
                plsc.store_scatter(dst, [kvec, col], vals * SCALE)
            return carry

        lax.fori_loop(0, D_MODEL, per_d, 0, unroll=4)

    def step(k, ab, first, load_next):
        wait_load(ab)
        if not first:
            wait_store(ab)
        transpose_slab(a_v.at[ab], b_v.at[ab], FV)
        start_store(k, ab)
        if load_next:
            start_load(k + 2, ab)

    start_load(0, 0)
    start_load(1, 1)
    step(0, 0, True, True)
    step(1, 1, True, True)

    def pair_iter(i, carry):
        step(2 * i + 2, 0, False, True)
        step(2 * i + 3, 1, False, True)
        return carry

    lax.fori_loop(0, (F_STEPS - 5) // 2, pair_iter, 0)
    step(F_STEPS - 3, 0, False, True)
    step(F_STEPS - 2, 1, False, False)
    step(F_STEPS - 1, 0, False, False)
    wait_store(0)
    wait_store(1)

    @pl.when(wid == F_EXTRA)
    def _tail():
        pltpu.sync_copy(tail_hbm, t_v)
        transpose_slab(t_v, b_v.at[0], F_TAIL)
        pltpu.async_copy(b_v.at[0, pl.ds(0, F_TAIL // 2)],
                         tab_hbm.at[pl.ds(N_FULL * FV // 2, F_TAIL // 2)],
                         ssems[0])
        pltpu.make_async_copy(b_v.at[0, pl.ds(0, F_TAIL // 2)],
                              tab_hbm.at[pl.ds(0, F_TAIL // 2)],
                              ssems[0]).wait()


def _gather_body(xt_hbm, tab_hbm, out_hbm, idx_v, p_v, h_v, pair_v, tr_v,
                 gsems, ssems):
    c = lax.axis_index("c")
    s = lax.axis_index("s")
    wid = s * 2 + c
    i0 = wid * CHUNK

    iota = _iota16()

    pltpu.sync_copy(xt_hbm.at[:, pl.ds(i0, CHUNK)], idx_v)

    def prep_and_gather(j, b):
        def prep(q, carry):
            xq = idx_v[j, pl.ds(LANES * q, LANES)]
            p_v[b, pl.ds(LANES * q, LANES)] = xq >> 1
            h_v[b, pl.ds(LANES * q, LANES)] = (xq & 1) << 6
            return carry

        lax.fori_loop(0, CHUNK // LANES, prep, 0, unroll=8)
        pltpu.async_copy(tab_hbm.at[p_v.at[b]], pair_v.at[b], gsems[b])

    def wait_gather(b):
        pltpu.make_async_copy(tab_hbm.at[p_v.at[0]], pair_v.at[b],
                              gsems[b]).wait()

    def start_store(j, t):
        pltpu.async_copy(tr_v.at[t], out_hbm.at[j, :, pl.ds(i0, CHUNK)],
                         ssems[t])

    def wait_store(t):
        pltpu.make_async_copy(tr_v.at[t], out_hbm.at[0, :, pl.ds(i0, CHUNK)],
                              ssems[t]).wait()

    def transpose_select(b, t):
        def per_r(r0, carry):
            for q in range(D_MODEL // LANES):
                dq = iota + (LANES * q)
                rr = (r0 + dq) & 127
                h64 = plsc.load_gather(h_v.at[b], [rr])
                vals = plsc.load_gather(pair_v.at[b], [rr, h64 + dq])
                plsc.store_scatter(tr_v.at[t], [dq, rr], vals)
            return carry

        lax.fori_loop(0, CHUNK, per_r, 0, unroll=4)

    for j in range(NBUF):
        prep_and_gather(j, j)
    for j in range(LEAD):
        wait_gather(j)
        transpose_select(j, j % TBUF)
        start_store(j, j % TBUF)

    def ring_iter(i, carry):
        for b in range(NBUF):
            j = i * NBUF + b + LEAD
            bj = (b + LEAD) % NBUF
            prep_and_gather(j + LEAD, b)
            wait_gather(bj)
            t = b % TBUF
            wait_store(t)
            transpose_select(bj, t)
            start_store(j, t)
        return carry

    lax.fori_loop(0, (N_J - 2 * LEAD) // NBUF, ring_iter, 0)

    for jj in range(N_J - LEAD, N_J):
        b = jj % NBUF
        wait_gather(b)
        t = jj % TBUF
        wait_store(t)
        transpose_select(b, t)
        start_store(jj, t)

    for t in range(TBUF):
        wait_store(t)


@jax.jit
def _embed(xt, lutt, tail):
    mesh = plsc.VectorSubcoreMesh(core_axis_name="c", subcore_axis_name="s")
    fmt = functools.partial(
        pl.kernel,
        out_type=jax.ShapeDtypeStruct((VOCAB // 2, 128), jnp.float32),
        mesh=mesh,
        compiler_params=pltpu.CompilerParams(needs_layout_passes=False),
        scratch_types=[
            pltpu.VMEM((2, D_MODEL, FV), jnp.float32),
            pltpu.VMEM((2, FV // 2, 128), jnp.float32),
            pltpu.VMEM((D_MODEL, F_TAIL), jnp.float32),
            [pltpu.SemaphoreType.DMA] * 2,
            [pltpu.SemaphoreType.DMA] * 2,
        ],
    )(_format_body)
    tab = fmt(lutt, tail)

    gat = functools.partial(
        pl.kernel,
        out_type=jax.ShapeDtypeStruct((N_J, D_MODEL, N_I), jnp.float32),
        mesh=mesh,
        compiler_params=pltpu.CompilerParams(needs_layout_passes=False),
        scratch_types=[
            pltpu.VMEM((N_J, CHUNK), jnp.int32),
            pltpu.VMEM((NBUF, CHUNK), jnp.int32),
            pltpu.VMEM((NBUF, CHUNK), jnp.int32),
            pltpu.VMEM((NBUF, CHUNK, 128), jnp.float32),
            pltpu.VMEM((TBUF, D_MODEL, CHUNK), jnp.float32),
            [pltpu.SemaphoreType.DMA] * NBUF,
            [pltpu.SemaphoreType.DMA] * TBUF,
        ],
    )(_gather_body)
    return gat(xt, tab)


def kernel(x, lut):
    xt = x.astype(jnp.int32).T
    lutt = lut.T
    tail = lutt[:, N_FULL * FV:]
    out_t = _embed(xt, lutt, tail)
    return out_t.transpose(2, 0, 1)


# speedup vs baseline: 4.4558x; 3.9960x over previous
import functools

import jax
import jax.numpy as jnp
from jax import lax
from jax.experimental import pallas as pl
from jax.experimental.pallas import tpu as pltpu
from jax.experimental.pallas import tpu_sc as plsc

D_MODEL = 64
VOCAB = 1000000
N_I = 4096
N_J = 200
NUM_WORKERS = 32
LANES = 16
SCALE = 8.0

FV = 256
N_FULL = VOCAB // FV
F_TAIL = VOCAB - N_FULL * FV
F_BASE = N_FULL // NUM_WORKERS
F_EXTRA = N_FULL - F_BASE * NUM_WORKERS
F_STEPS = F_BASE + 1

CHUNK = 128
NBUF = 4
TBUF = 2
LEAD = 2


def _iota16():
    return lax.broadcasted_iota(jnp.int32, (LANES,), 0)


def _format_body(lutt_hbm, tail_hbm, tab_hbm, a_v, b_v, t_v, lsems, ssems):
    c = lax.axis_index("c")
    s = lax.axis_index("s")
    wid = s * 2 + c
    base = wid * F_BASE + jnp.minimum(wid, F_EXTRA)

    iota = _iota16()
    hcol = (iota & 1) << 6

    def clamp(k):
        return jnp.minimum(base + k, N_FULL - 1)

    def start_load(k, ab):
        pltpu.async_copy(lutt_hbm.at[:, pl.ds(clamp(k) * FV, FV)],
                         a_v.at[ab], lsems[ab])

    def wait_load(ab):
        pltpu.make_async_copy(lutt_hbm.at[:, pl.ds(0, FV)], a_v.at[ab],
                              lsems[ab]).wait()

    def start_store(k, ab):
        pltpu.async_copy(b_v.at[ab],
                         tab_hbm.at[pl.ds(clamp(k) * (FV // 2), FV // 2)],
                         ssems[ab])

    def wait_store(ab):
        pltpu.make_async_copy(b_v.at[ab], tab_hbm.at[pl.ds(0, FV // 2)],
                              ssems[ab]).wait()

    def transpose_slab(src, dst, nv):
        @plsc.parallel_loop(0, D_MODEL, unroll=4)
        def per_d(d0):
            dvec = (d0 + iota) & 63
            col = hcol + dvec
            for g in range(nv // LANES):
                vvec = iota + (LANES * g)
                kvec = (iota >> 1) + (8 * g)
                vals = plsc.load_gather(src, [dvec, vvec])
                plsc.store_scatter(dst, [kvec, col], vals * SCALE)

    def step(k, ab, first, load_next):
        wait_load(ab)
        if not first:
            wait_store(ab)
        transpose_slab(a_v.at[ab], b_v.at[ab], FV)
        start_store(k, ab)
        if load_next:
            start_load(k + 2, ab)

    start_load(0, 0)
    start_load(1, 1)
    step(0, 0, True, True)
    step(1, 1, True, True)

    def pair_iter(i, carry):
        step(2 * i + 2, 0, False, True)
        step(2 * i + 3, 1, False, True)
        return carry

    lax.fori_loop(0, (F_STEPS - 5) // 2, pair_iter, 0)
    step(F_STEPS - 3, 0, False, True)
    step(F_STEPS - 2, 1, False, False)
    step(F_STEPS - 1, 0, False, False)
    wait_store(0)
    wait_store(1)

    @pl.when(wid == F_EXTRA)
    def _tail():
        pltpu.sync_copy(tail_hbm, t_v)
        transpose_slab(t_v, b_v.at[0], F_TAIL)
        pltpu.async_copy(b_v.at[0, pl.ds(0, F_TAIL // 2)],
                         tab_hbm.at[pl.ds(N_FULL * FV // 2, F_TAIL // 2)],
                         ssems[0])
        pltpu.make_async_copy(b_v.at[0, pl.ds(0, F_TAIL // 2)],
                              tab_hbm.at[pl.ds(0, F_TAIL // 2)],
                              ssems[0]).wait()


def _gather_body(xt_hbm, tab_hbm, out_hbm, idx_v, p_v, h_v, pair_v, tr_v,
                 gsems, ssems):
    c = lax.axis_index("c")
    s = lax.axis_index("s")
    wid = s * 2 + c
    i0 = wid * CHUNK

    iota = _iota16()

    pltpu.sync_copy(xt_hbm.at[:, pl.ds(i0, CHUNK)], idx_v)

    def prep_and_gather(j, b):
        def prep(q, carry):
            xq = idx_v[j, pl.ds(LANES * q, LANES)]
            p_v[b, pl.ds(LANES * q, LANES)] = xq >> 1
            h_v[b, pl.ds(LANES * q, LANES)] = (xq & 1) << 6
            return carry

        lax.fori_loop(0, CHUNK // LANES, prep, 0, unroll=8)
        pltpu.async_copy(tab_hbm.at[p_v.at[b]], pair_v.at[b], gsems[b])

    def wait_gather(b):
        pltpu.make_async_copy(tab_hbm.at[p_v.at[0]], pair_v.at[b],
                              gsems[b]).wait()

    def start_store(j, t):
        pltpu.async_copy(tr_v.at[t], out_hbm.at[j, :, pl.ds(i0, CHUNK)],
                         ssems[t])

    def wait_store(t):
        pltpu.make_async_copy(tr_v.at[t], out_hbm.at[0, :, pl.ds(i0, CHUNK)],
                              ssems[t]).wait()

    def transpose_select(b, t):
        @plsc.parallel_loop(0, CHUNK, unroll=4)
        def per_r(r0):
            for q in range(D_MODEL // LANES):
                dq = iota + (LANES * q)
                rr = (r0 + dq) & 127
                h64 = plsc.load_gather(h_v.at[b], [rr])
                vals = plsc.load_gather(pair_v.at[b], [rr, h64 + dq])
                plsc.store_scatter(tr_v.at[t], [dq, rr], vals)

    for j in range(NBUF):
        prep_and_gather(j, j)
    for j in range(LEAD):
        wait_gather(j)
        transpose_select(j, j % TBUF)
        start_store(j, j % TBUF)

    def ring_iter(i, carry):
        for b in range(NBUF):
            j = i * NBUF + b + LEAD
            bj = (b + LEAD) % NBUF
            prep_and_gather(j + LEAD, b)
            wait_gather(bj)
            t = b % TBUF
            wait_store(t)
            transpose_select(bj, t)
            start_store(j, t)
        return carry

    lax.fori_loop(0, (N_J - 2 * LEAD) // NBUF, ring_iter, 0)

    for jj in range(N_J - LEAD, N_J):
        b = jj % NBUF
        wait_gather(b)
        t = jj % TBUF
        wait_store(t)
        transpose_select(b, t)
        start_store(jj, t)

    for t in range(TBUF):
        wait_store(t)


@jax.jit
def _embed(xt, lutt, tail):
    mesh = plsc.VectorSubcoreMesh(core_axis_name="c", subcore_axis_name="s")
    fmt = functools.partial(
        pl.kernel,
        out_type=jax.ShapeDtypeStruct((VOCAB // 2, 128), jnp.float32),
        mesh=mesh,
        compiler_params=pltpu.CompilerParams(needs_layout_passes=False),
        scratch_types=[
            pltpu.VMEM((2, D_MODEL, FV), jnp.float32),
            pltpu.VMEM((2, FV // 2, 128), jnp.float32),
            pltpu.VMEM((D_MODEL, F_TAIL), jnp.float32),
            [pltpu.SemaphoreType.DMA] * 2,
            [pltpu.SemaphoreType.DMA] * 2,
        ],
    )(_format_body)
    tab = fmt(lutt, tail)

    gat = functools.partial(
        pl.kernel,
        out_type=jax.ShapeDtypeStruct((N_J, D_MODEL, N_I), jnp.float32),
        mesh=mesh,
        compiler_params=pltpu.CompilerParams(needs_layout_passes=False),
        scratch_types=[
            pltpu.VMEM((N_J, CHUNK), jnp.int32),
            pltpu.VMEM((NBUF, CHUNK), jnp.int32),
            pltpu.VMEM((NBUF, CHUNK), jnp.int32),
            pltpu.VMEM((NBUF, CHUNK, 128), jnp.float32),
            pltpu.VMEM((TBUF, D_MODEL, CHUNK), jnp.float32),
            [pltpu.SemaphoreType.DMA] * NBUF,
            [pltpu.SemaphoreType.DMA] * TBUF,
        ],
    )(_gather_body)
    return gat(xt, tab)


def kernel(x, lut):
    xt = x.astype(jnp.int32).T
    lutt = lut.T
    tail = lutt[:, N_FULL * FV:]
    out_t = _embed(xt, lutt, tail)
    return out_t.transpose(2, 0, 1)
